# Initial kernel scaffold; baseline (speedup 1.0000x reference)
import jax
import jax.numpy as jnp
from jax.experimental import pallas as pl


def kernel(x, enc_w1, enc_b1, enc_w2, enc_b2, enc_w3, enc_b3, dec_w1, dec_b1, dec_w2, dec_b2, dec_w3, dec_b3, dec_w4, dec_b4, emb_w):
    raise NotImplementedError("write your pallas kernel here")


# speedup vs baseline: 4.1810x; 4.1810x over previous
import functools

import jax
import jax.numpy as jnp
from jax.experimental import pallas as pl
from jax.experimental.pallas import tpu as pltpu


def _vqvae_kernel(x_ref, w1, b1, w2, b2, w3, b3,
                  wscore, enorm, wgather,
                  dw1, db1, dw2, db2, dw3, db3, dw4, db4,
                  recon_ref, ze_ref, emb_ref, *, n_codes):
    h = jnp.maximum(x_ref[...] @ w1[...] + b1[...], 0.0)
    h = jnp.maximum(h @ w2[...] + b2[...], 0.0)
    h = h @ w3[...] + b3[...]
    ze_ref[...] = h

    merit = 2.0 * (h @ wscore[...]) - enorm[...]
    r, hdim = merit.shape
    m3 = merit.reshape(r, hdim // n_codes, n_codes)
    idx = jnp.argmax(m3, axis=-1)
    oh = (idx[:, :, None]
          == jax.lax.broadcasted_iota(jnp.int32, m3.shape, 2)).astype(jnp.float32)
    q = oh.reshape(r, hdim) @ wgather[...]
    emb_ref[...] = q

    d = jnp.maximum(q @ dw1[...] + db1[...], 0.0)
    d = jnp.maximum(d @ dw2[...] + db2[...], 0.0)
    d = jnp.maximum(d @ dw3[...] + db3[...], 0.0)
    recon_ref[...] = jax.nn.sigmoid(d @ dw4[...] + db4[...])


def kernel(x, enc_w1, enc_b1, enc_w2, enc_b2, enc_w3, enc_b3,
           dec_w1, dec_b1, dec_w2, dec_b2, dec_w3, dec_b3, dec_w4, dec_b4,
           emb_w):
    bsz, lin = x.shape
    hdim = enc_w3.shape[0]
    kdim, ncodes = emb_w.shape
    seg = hdim // kdim

    eye_s = jnp.eye(seg, dtype=jnp.float32)
    wscore = (emb_w[:, None, None, :] * eye_s[None, :, :, None]
              ).reshape(kdim * seg, seg * ncodes)
    wgather = (eye_s[:, None, None, :] * emb_w.T[None, :, :, None]
               ).reshape(seg * ncodes, kdim * seg)
    enorm = jnp.tile(jnp.sum(emb_w * emb_w, axis=0), seg).reshape(1, hdim)

    blk = 512
    grid = (bsz // blk,)

    def row_spec(width):
        return pl.BlockSpec((blk, width), lambda i: (i, 0))

    def full_spec(a):
        return pl.BlockSpec(a.shape, lambda i: (0,) * a.ndim)

    weights = [enc_w1.T, enc_b1.reshape(1, -1), enc_w2.T, enc_b2.reshape(1, -1),
               enc_w3.T, enc_b3.reshape(1, -1),
               wscore, enorm, wgather,
               dec_w1.T, dec_b1.reshape(1, -1), dec_w2.T, dec_b2.reshape(1, -1),
               dec_w3.T, dec_b3.reshape(1, -1), dec_w4.T, dec_b4.reshape(1, -1)]

    recon, ze, emb_out = pl.pallas_call(
        functools.partial(_vqvae_kernel, n_codes=ncodes),
        grid=grid,
        in_specs=[row_spec(lin)] + [full_spec(w) for w in weights],
        out_specs=[row_spec(lin), row_spec(hdim), row_spec(hdim)],
        out_shape=[jax.ShapeDtypeStruct((bsz, lin), jnp.float32),
                   jax.ShapeDtypeStruct((bsz, hdim), jnp.float32),
                   jax.ShapeDtypeStruct((bsz, hdim), jnp.float32)],
        compiler_params=pltpu.CompilerParams(
            dimension_semantics=("parallel",)),
    )(x, *weights)

    return recon, ze.reshape(bsz, kdim, seg), emb_out


# speedup vs baseline: 4.7325x; 1.1319x over previous
import functools

import jax
import jax.numpy as jnp
from jax.experimental import pallas as pl
from jax.experimental.pallas import tpu as pltpu

_BF = jnp.bfloat16
_F32 = jnp.float32


def _mm(a, b):
    return jax.lax.dot(a.astype(_BF), b.astype(_BF),
                       preferred_element_type=_F32)


def _vqvae_kernel(x_ref, w1, b1, w2, b2, w3, b3,
                  wsc, enorm2, wgt,
                  dw1, db1, dw2, db2, dw3, db3, dw4, db4,
                  recon_ref, ze_ref, emb_ref, *, n_codes):
    h = jnp.maximum(x_ref[...] @ w1[...] + b1[...], 0.0)
    h = jnp.maximum(h @ w2[...] + b2[...], 0.0)
    h = h @ w3[...] + b3[...]
    ze_ref[...] = h

    merits = [2.0 * (h @ wsc[n]) - enorm2[n] for n in range(n_codes)]
    best = merits[0]
    bidx = jnp.zeros_like(best, dtype=jnp.int32)
    for n in range(1, n_codes):
        upd = merits[n] > best
        best = jnp.where(upd, merits[n], best)
        bidx = jnp.where(upd, n, bidx)

    q = _mm((bidx == 0).astype(_BF), wgt[0])
    for n in range(1, n_codes):
        q = q + _mm((bidx == n).astype(_BF), wgt[n])
    emb_ref[...] = q

    d = jnp.maximum(_mm(q, dw1[...]) + db1[...], 0.0)
    d = jnp.maximum(_mm(d, dw2[...]) + db2[...], 0.0)
    d = jnp.maximum(_mm(d, dw3[...]) + db3[...], 0.0)
    recon_ref[...] = jax.nn.sigmoid(_mm(d, dw4[...]) + db4[...])


def kernel(x, enc_w1, enc_b1, enc_w2, enc_b2, enc_w3, enc_b3,
           dec_w1, dec_b1, dec_w2, dec_b2, dec_w3, dec_b3, dec_w4, dec_b4,
           emb_w):
    bsz, lin = x.shape
    hdim = enc_w3.shape[0]
    kdim, ncodes = emb_w.shape
    seg = hdim // kdim

    eye_s = jnp.eye(seg, dtype=jnp.float32)
    wsc = (emb_w.T[:, :, None, None] * eye_s[None, None, :, :]
           ).reshape(ncodes, kdim * seg, seg)
    wgt = (emb_w.T[:, None, :, None] * eye_s[None, :, None, :]
           ).reshape(ncodes, seg, kdim * seg).astype(_BF)
    enorm2 = jnp.sum(emb_w * emb_w, axis=0).reshape(ncodes, 1, 1)

    blk = 512
    grid = (bsz // blk,)

    def row_spec(width):
        return pl.BlockSpec((blk, width), lambda i: (i, 0))

    def full_spec(a):
        return pl.BlockSpec(a.shape, lambda i: (0,) * a.ndim)

    weights = [enc_w1.T, enc_b1.reshape(1, -1), enc_w2.T, enc_b2.reshape(1, -1),
               enc_w3.T, enc_b3.reshape(1, -1),
               wsc, enorm2, wgt,
               dec_w1.T.astype(_BF), dec_b1.reshape(1, -1),
               dec_w2.T.astype(_BF), dec_b2.reshape(1, -1),
               dec_w3.T.astype(_BF), dec_b3.reshape(1, -1),
               dec_w4.T.astype(_BF), dec_b4.reshape(1, -1)]

    recon, ze, emb_out = pl.pallas_call(
        functools.partial(_vqvae_kernel, n_codes=ncodes),
        grid=grid,
        in_specs=[row_spec(lin)] + [full_spec(w) for w in weights],
        out_specs=[row_spec(lin), row_spec(hdim), row_spec(hdim)],
        out_shape=[jax.ShapeDtypeStruct((bsz, lin), jnp.float32),
                   jax.ShapeDtypeStruct((bsz, hdim), jnp.float32),
                   jax.ShapeDtypeStruct((bsz, hdim), jnp.float32)],
        compiler_params=pltpu.CompilerParams(
            dimension_semantics=("parallel",)),
    )(x, *weights)

    return recon, ze.reshape(bsz, kdim, seg), emb_out


# speedup vs baseline: 4.8699x; 1.0290x over previous
import functools

import jax
import jax.numpy as jnp
from jax.experimental import pallas as pl
from jax.experimental.pallas import tpu as pltpu

_BF = jnp.bfloat16
_F32 = jnp.float32


def _mm(a, b):
    return jax.lax.dot(a.astype(_BF), b.astype(_BF),
                       preferred_element_type=_F32)


def _vqvae_kernel(x_ref, w1, b1, w2, b2, w3, b3,
                  wsc, enorm2, wgt,
                  dw1, db1, dw2, db2, dw3, db3, dw4, db4,
                  recon_ref, ze_ref, emb_ref, *, n_codes):
    h = jnp.maximum(x_ref[...] @ w1[...] + b1[...], 0.0)
    h = jnp.maximum(h @ w2[...] + b2[...], 0.0)
    h = h @ w3[...] + b3[...]
    ze_ref[...] = h

    merits = [2.0 * (h @ wsc[n]) - enorm2[n] for n in range(n_codes)]
    best = merits[0]
    bidx = jnp.zeros_like(best, dtype=jnp.int32)
    for n in range(1, n_codes):
        upd = merits[n] > best
        best = jnp.where(upd, merits[n], best)
        bidx = jnp.where(upd, n, bidx)

    q = _mm((bidx == 0).astype(_BF), wgt[0])
    for n in range(1, n_codes):
        q = q + _mm((bidx == n).astype(_BF), wgt[n])
    emb_ref[...] = q

    d = jnp.maximum(_mm(q, dw1[...]) + db1[...], 0.0)
    d = jnp.maximum(_mm(d, dw2[...]) + db2[...], 0.0)
    d = jnp.maximum(_mm(d, dw3[...]) + db3[...], 0.0)
    recon_ref[...] = jax.nn.sigmoid(_mm(d, dw4[...]) + db4[...])


def kernel(x, enc_w1, enc_b1, enc_w2, enc_b2, enc_w3, enc_b3,
           dec_w1, dec_b1, dec_w2, dec_b2, dec_w3, dec_b3, dec_w4, dec_b4,
           emb_w):
    bsz, lin = x.shape
    hdim = enc_w3.shape[0]
    kdim, ncodes = emb_w.shape
    seg = hdim // kdim

    eye_s = jnp.eye(seg, dtype=jnp.float32)
    wsc = (emb_w.T[:, :, None, None] * eye_s[None, None, :, :]
           ).reshape(ncodes, kdim * seg, seg)
    wgt = (emb_w.T[:, None, :, None] * eye_s[None, :, None, :]
           ).reshape(ncodes, seg, kdim * seg).astype(_BF)
    enorm2 = jnp.sum(emb_w * emb_w, axis=0).reshape(ncodes, 1, 1)

    blk = 1024
    grid = (bsz // blk,)

    def row_spec(width):
        return pl.BlockSpec((blk, width), lambda i: (i, 0))

    def full_spec(a):
        return pl.BlockSpec(a.shape, lambda i: (0,) * a.ndim)

    weights = [enc_w1.T, enc_b1.reshape(1, -1), enc_w2.T, enc_b2.reshape(1, -1),
               enc_w3.T, enc_b3.reshape(1, -1),
               wsc, enorm2, wgt,
               dec_w1.T.astype(_BF), dec_b1.reshape(1, -1),
               dec_w2.T.astype(_BF), dec_b2.reshape(1, -1),
               dec_w3.T.astype(_BF), dec_b3.reshape(1, -1),
               dec_w4.T.astype(_BF), dec_b4.reshape(1, -1)]

    recon, ze, emb_out = pl.pallas_call(
        functools.partial(_vqvae_kernel, n_codes=ncodes),
        grid=grid,
        in_specs=[row_spec(lin)] + [full_spec(w) for w in weights],
        out_specs=[row_spec(lin), row_spec(hdim), row_spec(hdim)],
        out_shape=[jax.ShapeDtypeStruct((bsz, lin), jnp.float32),
                   jax.ShapeDtypeStruct((bsz, hdim), jnp.float32),
                   jax.ShapeDtypeStruct((bsz, hdim), jnp.float32)],
        compiler_params=pltpu.CompilerParams(
            dimension_semantics=("parallel",)),
    )(x, *weights)

    return recon, ze.reshape(bsz, kdim, seg), emb_out


# speedup vs baseline: 4.9350x; 1.0134x over previous
import functools

import jax
import jax.numpy as jnp
from jax.experimental import pallas as pl
from jax.experimental.pallas import tpu as pltpu

_BF = jnp.bfloat16
_F32 = jnp.float32


def _mm(a, b):
    return jax.lax.dot(a.astype(_BF), b.astype(_BF),
                       preferred_element_type=_F32)


def _vqvae_kernel(x_ref, w1, b1, w2, b2, w3, b3,
                  wsc, enorm2, wgt,
                  dw1, db1, dw2, db2, dw3, db3, dw4, db4,
                  recon_ref, ze_ref, emb_ref, *, n_codes):
    h = jnp.maximum(x_ref[...] @ w1[...] + b1[...], 0.0)
    h = jnp.maximum(h @ w2[...] + b2[...], 0.0)
    h = h @ w3[...] + b3[...]
    ze_ref[...] = h

    merits = [2.0 * (h @ wsc[n]) - enorm2[n] for n in range(n_codes)]
    best = merits[0]
    bidx = jnp.zeros_like(best, dtype=jnp.int32)
    for n in range(1, n_codes):
        upd = merits[n] > best
        best = jnp.where(upd, merits[n], best)
        bidx = jnp.where(upd, n, bidx)

    q = _mm((bidx == 0).astype(_BF), wgt[0])
    for n in range(1, n_codes):
        q = q + _mm((bidx == n).astype(_BF), wgt[n])
    emb_ref[...] = q

    d = jnp.maximum(_mm(q, dw1[...]) + db1[...], 0.0)
    d = jnp.maximum(_mm(d, dw2[...]) + db2[...], 0.0)
    d = jnp.maximum(_mm(d, dw3[...]) + db3[...], 0.0)
    recon_ref[...] = jax.nn.sigmoid(_mm(d, dw4[...]) + db4[...])


def kernel(x, enc_w1, enc_b1, enc_w2, enc_b2, enc_w3, enc_b3,
           dec_w1, dec_b1, dec_w2, dec_b2, dec_w3, dec_b3, dec_w4, dec_b4,
           emb_w):
    bsz, lin = x.shape
    hdim = enc_w3.shape[0]
    kdim, ncodes = emb_w.shape
    seg = hdim // kdim

    eye_s = jnp.eye(seg, dtype=jnp.float32)
    wsc = (emb_w.T[:, :, None, None] * eye_s[None, None, :, :]
           ).reshape(ncodes, kdim * seg, seg)
    wgt = (emb_w.T[:, None, :, None] * eye_s[None, :, None, :]
           ).reshape(ncodes, seg, kdim * seg).astype(_BF)
    enorm2 = jnp.sum(emb_w * emb_w, axis=0).reshape(ncodes, 1, 1)

    blk = 2048
    grid = (bsz // blk,)

    def row_spec(width):
        return pl.BlockSpec((blk, width), lambda i: (i, 0))

    def full_spec(a):
        return pl.BlockSpec(a.shape, lambda i: (0,) * a.ndim)

    weights = [enc_w1.T, enc_b1.reshape(1, -1), enc_w2.T, enc_b2.reshape(1, -1),
               enc_w3.T, enc_b3.reshape(1, -1),
               wsc, enorm2, wgt,
               dec_w1.T.astype(_BF), dec_b1.reshape(1, -1),
               dec_w2.T.astype(_BF), dec_b2.reshape(1, -1),
               dec_w3.T.astype(_BF), dec_b3.reshape(1, -1),
               dec_w4.T.astype(_BF), dec_b4.reshape(1, -1)]

    recon, ze, emb_out = pl.pallas_call(
        functools.partial(_vqvae_kernel, n_codes=ncodes),
        grid=grid,
        in_specs=[row_spec(lin)] + [full_spec(w) for w in weights],
        out_specs=[row_spec(lin), row_spec(hdim), row_spec(hdim)],
        out_shape=[jax.ShapeDtypeStruct((bsz, lin), jnp.float32),
                   jax.ShapeDtypeStruct((bsz, hdim), jnp.float32),
                   jax.ShapeDtypeStruct((bsz, hdim), jnp.float32)],
        compiler_params=pltpu.CompilerParams(
            dimension_semantics=("parallel",)),
    )(x, *weights)

    return recon, ze.reshape(bsz, kdim, seg), emb_out


# speedup vs baseline: 4.9381x; 1.0006x over previous
import functools

import jax
import jax.numpy as jnp
from jax.experimental import pallas as pl
from jax.experimental.pallas import tpu as pltpu

_BF = jnp.bfloat16
_F32 = jnp.float32


def _mm(a, b):
    return jax.lax.dot(a.astype(_BF), b.astype(_BF),
                       preferred_element_type=_F32)


def _vqvae_kernel(x_ref, w1, b1, w2, b2, w3, b3,
                  wsc, enorm2, wgt,
                  dw1, db1, dw2, db2, dw3, db3, dw4, db4,
                  recon_ref, ze_ref, emb_ref, *, n_codes):
    h = jnp.maximum(x_ref[...] @ w1[...] + b1[...], 0.0)
    h = jnp.maximum(h @ w2[...] + b2[...], 0.0)
    h = h @ w3[...] + b3[...]
    ze_ref[...] = h

    merits = [2.0 * (h @ wsc[n]) - enorm2[n] for n in range(n_codes)]
    best = merits[0]
    bidx = jnp.zeros_like(best, dtype=jnp.int32)
    for n in range(1, n_codes):
        upd = merits[n] > best
        best = jnp.where(upd, merits[n], best)
        bidx = jnp.where(upd, n, bidx)

    q = _mm((bidx == 0).astype(_BF), wgt[0])
    for n in range(1, n_codes):
        q = q + _mm((bidx == n).astype(_BF), wgt[n])
    emb_ref[...] = q

    d = jnp.maximum(_mm(q, dw1[...]) + db1[...], 0.0)
    d = jnp.maximum(_mm(d, dw2[...]) + db2[...], 0.0)
    d = jnp.maximum(_mm(d, dw3[...]) + db3[...], 0.0)
    recon_ref[...] = jax.nn.sigmoid(_mm(d, dw4[...]) + db4[...])


def kernel(x, enc_w1, enc_b1, enc_w2, enc_b2, enc_w3, enc_b3,
           dec_w1, dec_b1, dec_w2, dec_b2, dec_w3, dec_b3, dec_w4, dec_b4,
           emb_w):
    bsz, lin = x.shape
    hdim = enc_w3.shape[0]
    kdim, ncodes = emb_w.shape
    seg = hdim // kdim

    eye_s = jnp.eye(seg, dtype=jnp.float32)
    wsc = (emb_w.T[:, :, None, None] * eye_s[None, None, :, :]
           ).reshape(ncodes, kdim * seg, seg)
    wgt = (emb_w.T[:, None, :, None] * eye_s[None, :, None, :]
           ).reshape(ncodes, seg, kdim * seg).astype(_BF)
    enorm2 = jnp.sum(emb_w * emb_w, axis=0).reshape(ncodes, 1, 1)

    blk = 2048
    grid = (bsz // blk,)

    def row_spec(width):
        return pl.BlockSpec((blk, width), lambda i: (i, 0))

    def full_spec(a):
        return pl.BlockSpec(a.shape, lambda i: (0,) * a.ndim)

    weights = [enc_w1.T, enc_b1.reshape(1, -1), enc_w2.T, enc_b2.reshape(1, -1),
               enc_w3.T, enc_b3.reshape(1, -1),
               wsc, enorm2, wgt,
               dec_w1.T.astype(_BF), dec_b1.reshape(1, -1),
               dec_w2.T.astype(_BF), dec_b2.reshape(1, -1),
               dec_w3.T.astype(_BF), dec_b3.reshape(1, -1),
               dec_w4.T.astype(_BF), dec_b4.reshape(1, -1)]

    recon, ze, emb_out = pl.pallas_call(
        functools.partial(_vqvae_kernel, n_codes=ncodes),
        grid=grid,
        in_specs=[row_spec(lin)] + [full_spec(w) for w in weights],
        out_specs=[row_spec(lin), row_spec(hdim), row_spec(hdim)],
        out_shape=[jax.ShapeDtypeStruct((bsz, lin), jnp.float32),
                   jax.ShapeDtypeStruct((bsz, hdim), jnp.float32),
                   jax.ShapeDtypeStruct((bsz, hdim), jnp.float32)],
        compiler_params=pltpu.CompilerParams(
            dimension_semantics=("arbitrary",)),
    )(x, *weights)

    return recon, ze.reshape(bsz, kdim, seg), emb_out
